# baseline (speedup 1.0000x reference)
import functools

import jax
import jax.numpy as jnp
from jax import lax
from jax.experimental import pallas as pl
from jax.experimental.pallas import tpu as pltpu
from jax.experimental.pallas import tpu_sc as plsc

_NC = 2
_NS = 16
_L = 16


def _sc_scatter_partials(idx1, upd1, Mp):
    n = idx1.shape[0]
    NW = _NC * _NS
    B = 8192
    per_w = n // NW
    nblk = per_w // B
    ZB = 16384
    per_s = Mp // _NS

    mesh = plsc.VectorSubcoreMesh(
        core_axis_name="c", subcore_axis_name="s",
        num_cores=_NC, num_subcores=_NS)

    @functools.partial(
        pl.kernel,
        out_type=[jax.ShapeDtypeStruct((Mp,), jnp.float32),
                  jax.ShapeDtypeStruct((Mp,), jnp.float32)],
        mesh=mesh,
        scratch_types=[
            pltpu.VMEM((B,), jnp.int32),
            pltpu.VMEM((B,), jnp.float32),
            pltpu.VMEM((B,), jnp.int32),
            pltpu.VMEM((B,), jnp.float32),
            pltpu.VMEM((ZB,), jnp.float32),
            pltpu.VMEM_SHARED((Mp,), jnp.float32),
            pltpu.SemaphoreType.DMA,
            pltpu.SemaphoreType.DMA,
        ],
    )
    def k(idx_hbm, upd_hbm, out0, out1, idx_a, upd_a, idx_b,
          upd_b, zero_v, acc, sem_a, sem_b):
        c = lax.axis_index("c")
        s = lax.axis_index("s")
        w = c * _NS + s

        bufs = ((idx_a, upd_a, sem_a), (idx_b, upd_b, sem_b))

        def start(b, iv, uv, sem):
            base = (w * nblk + b) * B
            ci = pltpu.async_copy(idx_hbm.at[pl.ds(base, B)], iv, sem)
            cu = pltpu.async_copy(upd_hbm.at[pl.ds(base, B)], uv, sem)
            return ci, cu

        pend = start(0, *bufs[0])
        import contextlib
        scope = jax.named_scope

        with scope("ph1_zero"):
            def zstore(i, carry):
                zero_v[pl.ds(i * _L, _L)] = jnp.zeros((_L,), jnp.float32)
                return carry
            lax.fori_loop(0, ZB // _L, zstore, 0)

            def zdma(i, carry):
                pltpu.sync_copy(zero_v, acc.at[pl.ds(s * per_s + i * ZB, ZB)])
                return carry
            lax.fori_loop(0, per_s // ZB, zdma, 0)

            plsc.subcore_barrier()

        with scope("ph2_scatter"):
            for b in range(nblk):
                iv, uv, _ = bufs[b % 2]
                pend[0].wait()
                pend[1].wait()
                if b + 1 < nblk:
                    pend = start(b + 1, *bufs[(b + 1) % 2])
                pltpu.sync_copy(uv, acc.at[iv], add=True)
            plsc.subcore_barrier()

        scope3 = scope("ph3_writeout"); scope3.__enter__()
        @pl.when(c == 0)
        def _():
            pltpu.sync_copy(acc.at[pl.ds(s * per_s, per_s)],
                            out0.at[pl.ds(s * per_s, per_s)])

        @pl.when(c == 1)
        def _():
            pltpu.sync_copy(acc.at[pl.ds(s * per_s, per_s)],
                            out1.at[pl.ds(s * per_s, per_s)])
        scope3.__exit__(None, None, None)

    return k(idx1, upd1)


def _combine(d, a, b):
    M = d.shape[0]
    Mp = a.shape[0]
    BLK = Mp // 8

    def body(d_ref, a_ref, b_ref, o_ref):
        o_ref[...] = d_ref[...] + a_ref[...] + b_ref[...]

    return pl.pallas_call(
        body,
        grid=(Mp // BLK,),
        in_specs=[pl.BlockSpec((BLK,), lambda i: (i,))] * 3,
        out_specs=pl.BlockSpec((BLK,), lambda i: (i,)),
        out_shape=jax.ShapeDtypeStruct((M,), jnp.float32),
    )(d, a, b)


def kernel(data, indices, updates):
    Mp = 1 << 20
    q0, q1 = _sc_scatter_partials(indices.astype(jnp.int32), updates, Mp)
    return _combine(data, q0, q1)
